# baseline (speedup 1.0000x reference)
import functools

import jax
import jax.numpy as jnp
from jax import lax
from jax.experimental import pallas as pl
from jax.experimental.pallas import tpu as pltpu
from jax.experimental.pallas import tpu_sc as plsc

N_IN = 10000
N_PAD = 10112
M_OUT = 10000
M_PAD = 10240
KN = 32
MB = 128
F3 = 27
CIN = 128
COUT = 128
NC, NS = 2, 16
NW = NC * NS
IDX_ROWS = M_PAD * KN // 128
ROWS_W = IDX_ROWS // NW


NG = N_PAD // MB
GK = 8
NKEY = NG * GK
NKEY_PAD = 640


def _topk_body(opt_ref, ipc_ref, idx_ref, d2_ref, dist_ref, kv_ref, kc_ref,
               g8_ref):
    ox = opt_ref[0:1, :]
    oy = opt_ref[1:2, :]
    oz = opt_ref[2:3, :]
    ix = ipc_ref[:, 0:1]
    iy = ipc_ref[:, 1:2]
    iz = ipc_ref[:, 2:3]
    oo = (ox * ox + oy * oy) + oz * oz
    ii = (ix * ix + iy * iy) + iz * iz
    ob = opt_ref[...].astype(jnp.bfloat16)
    ib = ipc_ref[...].astype(jnp.bfloat16)
    dot = jnp.dot(ib, ob, preferred_element_type=jnp.float32)
    dist_ref[...] = jnp.maximum((oo + ii) - 2.0 * dot, 0.0)

    kv_ref[pl.ds(NKEY, NKEY_PAD - NKEY), :] = jnp.full(
        (NKEY_PAD - NKEY, MB), jnp.inf, jnp.float32)
    g8_ref[pl.ds(NG, 1), :] = jnp.full((1, MB), jnp.inf, jnp.float32)
    riota0 = lax.broadcasted_iota(jnp.int32, (MB, MB), 0)

    def grp(g, carry):
        dg = dist_ref[pl.ds(g * MB, MB), :]
        riota = riota0 + g * MB
        vs, cs = [], []
        for t in range(GK):
            vmin = jnp.min(dg, axis=0, keepdims=True)
            sel = jnp.where(dg == vmin, riota, 0x7FFFFFFF)
            vidx = jnp.min(sel, axis=0, keepdims=True)
            vs.append(vmin)
            cs.append(vidx)
            if t < GK - 1:
                dg = jnp.where(riota == vidx, jnp.float32(jnp.inf), dg)
        kv_ref[pl.ds(g * GK, GK), :] = jnp.concatenate(vs, axis=0)
        kc_ref[pl.ds(g * GK, GK), :] = jnp.concatenate(cs, axis=0)
        g8_ref[pl.ds(g, 1), :] = vs[-1]
        return carry

    lax.fori_loop(0, NG, grp, 0)

    v32 = None
    vs, cs = [], []
    for i in range(KN):
        kvv = kv_ref[...]
        vmin = jnp.min(kvv, axis=0, keepdims=True)
        sel = jnp.where(kvv == vmin, kc_ref[...], 0x7FFFFFFF)
        vidx = jnp.min(sel, axis=0, keepdims=True)
        vs.append(vmin)
        cs.append(vidx)
        kv_ref[...] = jnp.where(kc_ref[...] == vidx, jnp.float32(jnp.inf),
                                kvv)
        if (i + 1) % 8 == 0:
            d2_ref[pl.ds(i - 7, 8), :] = jnp.concatenate(vs, axis=0)
            idx_ref[pl.ds(i - 7, 8), :] = jnp.concatenate(cs, axis=0)
            vs, cs = [], []
        v32 = vmin

    ming8 = jnp.min(g8_ref[...], axis=0, keepdims=True)
    need_full = jnp.sum((ming8 <= v32).astype(jnp.float32)) > 0.0

    @pl.when(need_full)
    def _fallback():
        riota_f = lax.broadcasted_iota(jnp.int32, (N_PAD, MB), 0)
        for i in range(KN):
            d = dist_ref[...]
            vmin = jnp.min(d, axis=0, keepdims=True)
            sel = jnp.where(d == vmin, riota_f, 0x7FFFFFFF)
            vidx = jnp.min(sel, axis=0, keepdims=True)
            d2_ref[i:i + 1, :] = vmin
            idx_ref[i:i + 1, :] = vidx
            dist_ref[...] = jnp.where(riota_f == vidx, jnp.float32(jnp.inf),
                                      d)


def _topk_call(op_t, ip_cols):
    return pl.pallas_call(
        _topk_body,
        grid=(M_PAD // MB,),
        in_specs=[
            pl.BlockSpec((8, MB), lambda j: (0, j)),
            pl.BlockSpec((N_PAD, 8), lambda j: (0, 0)),
        ],
        out_specs=[
            pl.BlockSpec((KN, MB), lambda j: (0, j)),
            pl.BlockSpec((KN, MB), lambda j: (0, j)),
        ],
        out_shape=[
            jax.ShapeDtypeStruct((KN, M_PAD), jnp.int32),
            jax.ShapeDtypeStruct((KN, M_PAD), jnp.float32),
        ],
        scratch_shapes=[
            pltpu.VMEM((N_PAD, MB), jnp.float32),
            pltpu.VMEM((NKEY_PAD, MB), jnp.float32),
            pltpu.VMEM((NKEY_PAD, MB), jnp.int32),
            pltpu.VMEM((NG + 1, MB), jnp.float32),
        ],
    )(op_t, ip_cols)


def _sc_gather_body(feat_ref, pos_ref, idx_ref, g_ref, p_ref,
                    idx_v, rows_v, pos_v, sem_f, sem_p):
    wid = lax.axis_index("s") * NC + lax.axis_index("c")

    def body(c, carry):
        r = wid * ROWS_W + c
        pltpu.sync_copy(idx_ref.at[r], idx_v)
        cf = pltpu.async_copy(feat_ref.at[idx_v], rows_v, sem_f)
        cp = pltpu.async_copy(pos_ref.at[idx_v], pos_v, sem_p)
        cf.wait()
        cp.wait()
        off = pl.multiple_of(r * 128, 128)
        pltpu.sync_copy(rows_v, g_ref.at[pl.ds(off, 128)])
        pltpu.sync_copy(pos_v, p_ref.at[pl.ds(off, 128)])
        return carry

    lax.fori_loop(0, ROWS_W, body, 0)


@functools.lru_cache(maxsize=1)
def _sc_gather_kernel():
    return functools.partial(
        pl.kernel,
        mesh=plsc.VectorSubcoreMesh(core_axis_name="c", subcore_axis_name="s"),
        compiler_params=pltpu.CompilerParams(use_tc_tiling_on_sc=False),
        out_type=[
            jax.ShapeDtypeStruct((M_PAD * KN, CIN), jnp.float32),
            jax.ShapeDtypeStruct((M_PAD * KN, 16), jnp.float32),
        ],
        scratch_types=[
            pltpu.VMEM((128,), jnp.int32),
            pltpu.VMEM((128, CIN), jnp.float32),
            pltpu.VMEM((128, 16), jnp.float32),
            pltpu.SemaphoreType.DMA,
            pltpu.SemaphoreType.DMA,
        ],
    )(_sc_gather_body)


def _conv_body(g_ref, px_ref, py_ref, pz_ref, d2_ref, op_ref, kf_ref, b_ref,
               r_ref, out_ref):
    r = r_ref[0, 0]
    relx = px_ref[...] - op_ref[:, 0:1]
    rely = py_ref[...] - op_ref[:, 1:2]
    relz = pz_ref[...] - op_ref[:, 2:3]
    pxn = relx / r
    pyn = rely / r
    pzn = relz / r
    l2 = jnp.sqrt(pxn * pxn + pyn * pyn + pzn * pzn + 1e-12)
    linf = jnp.maximum(jnp.maximum(jnp.abs(pxn), jnp.abs(pyn)), jnp.abs(pzn))
    scale = l2 / jnp.maximum(linf, 1e-8)
    big = linf > 1e-8
    yx = jnp.where(big, pxn * scale, pxn)
    yy = jnp.where(big, pyn * scale, pyn)
    yz = jnp.where(big, pzn * scale, pzn)

    def axis_taps(y):
        co = jnp.clip((y * 0.5 + 0.5) * 2.0, 0.0, 2.0)
        c0 = jnp.floor(co)
        fr = co - c0
        c0i = c0.astype(jnp.int32)
        c1i = jnp.minimum(c0i + 1, 2)
        w0 = 1.0 - fr
        return [
            jnp.where(c0i == t, w0, 0.0) + jnp.where(c1i == t, fr, 0.0)
            for t in (0, 1, 2)
        ]

    ax = axis_taps(yx)
    ay = axis_taps(yy)
    az = axis_taps(yz)
    vm = (d2_ref[...] <= r * r).astype(jnp.float32)
    az = [a * vm for a in az]

    acc = jnp.zeros((MB, COUT), jnp.float32)
    for f in range(F3):
        wf = ax[f // 9] * ay[(f // 3) % 3] * az[f % 3]
        u = jnp.zeros((MB, CIN), jnp.float32)
        for k in range(KN):
            u = u + wf[:, k:k + 1] * g_ref[:, k, :]
        acc = acc + jnp.dot(u, kf_ref[f * CIN:(f + 1) * CIN, :],
                            preferred_element_type=jnp.float32)

    cnt = jnp.sum(vm, axis=1, keepdims=True)
    inv = jnp.where(cnt > 0, 1.0 / jnp.maximum(cnt, 1.0), 0.0)
    out_ref[...] = acc * inv + b_ref[...]


def _conv_call(g3, px, py, pz, d2, op_pad, kflat, bias2, rad):
    return pl.pallas_call(
        _conv_body,
        grid=(M_PAD // MB,),
        in_specs=[
            pl.BlockSpec((MB, KN, CIN), lambda j: (j, 0, 0)),
            pl.BlockSpec((MB, KN), lambda j: (j, 0)),
            pl.BlockSpec((MB, KN), lambda j: (j, 0)),
            pl.BlockSpec((MB, KN), lambda j: (j, 0)),
            pl.BlockSpec((MB, KN), lambda j: (j, 0)),
            pl.BlockSpec((MB, 16), lambda j: (j, 0)),
            pl.BlockSpec((F3 * CIN, COUT), lambda j: (0, 0)),
            pl.BlockSpec((1, COUT), lambda j: (0, 0)),
            pl.BlockSpec(memory_space=pltpu.SMEM),
        ],
        out_specs=pl.BlockSpec((MB, COUT), lambda j: (j, 0)),
        out_shape=jax.ShapeDtypeStruct((M_PAD, COUT), jnp.float32),
    )(g3, px, py, pz, d2, op_pad, kflat, bias2, rad)


def kernel(inp_features, inp_positions, out_positions, extents, kernel, bias):
    f32 = jnp.float32
    radius = jnp.asarray(extents).astype(f32) * 0.5
    op_t = jnp.zeros((8, M_PAD), f32).at[0:3, :M_OUT].set(
        out_positions.T.astype(f32))
    ip_cols = jnp.zeros((N_PAD, 8), f32)
    ip_cols = ip_cols.at[:N_IN, 0:3].set(inp_positions.astype(f32))
    ip_cols = ip_cols.at[N_IN:, 0:3].set(1e4)
    op_pad = jnp.zeros((M_PAD, 16), f32).at[:M_OUT, 0:3].set(
        out_positions.astype(f32))
    idx_t, d2_t = _topk_call(op_t, ip_cols)
    idx = idx_t.T
    d2 = d2_t.T
    idx2d = idx.reshape(IDX_ROWS, 128)
    pos16 = jnp.zeros((N_IN, 16), f32).at[:, 0:3].set(
        inp_positions.astype(f32))
    g_flat, p_flat = _sc_gather_kernel()(
        inp_features.astype(f32), pos16, idx2d)
    g3 = g_flat.reshape(M_PAD, KN, CIN)
    px = p_flat[:, 0].reshape(M_PAD, KN)
    py = p_flat[:, 1].reshape(M_PAD, KN)
    pz = p_flat[:, 2].reshape(M_PAD, KN)
    kflat = kernel.reshape(F3 * CIN, COUT).astype(f32)
    bias2 = bias.reshape(1, COUT).astype(f32)
    rad = jnp.reshape(radius, (1, 1))
    out_full = _conv_call(g3, px, py, pz, d2, op_pad, kflat, bias2, rad)
    return out_full[:M_OUT]


# speedup vs baseline: 1.3960x; 1.3960x over previous
import functools

import jax
import jax.numpy as jnp
from jax import lax
from jax.experimental import pallas as pl
from jax.experimental.pallas import tpu as pltpu
from jax.experimental.pallas import tpu_sc as plsc

N_IN = 10000
N_PAD = 10112
M_OUT = 10000
M_PAD = 10240
KN = 32
MB = 128
F3 = 27
CIN = 128
COUT = 128
NC, NS = 2, 16
NW = NC * NS
IDX_ROWS = M_PAD * KN // 128
ROWS_W = IDX_ROWS // NW


NG = N_PAD // MB
GK = 8
NKEY = NG * GK
NKEY_PAD = 640


def _topk_body(opt_ref, ipc_ref, idx_ref, d2_ref, dist_ref, kv_ref, kc_ref,
               g8_ref):
    ox = opt_ref[0:1, :]
    oy = opt_ref[1:2, :]
    oz = opt_ref[2:3, :]
    ix = ipc_ref[:, 0:1]
    iy = ipc_ref[:, 1:2]
    iz = ipc_ref[:, 2:3]
    oo = (ox * ox + oy * oy) + oz * oz
    ii = (ix * ix + iy * iy) + iz * iz
    ob = opt_ref[...].astype(jnp.bfloat16)
    ib = ipc_ref[...].astype(jnp.bfloat16)
    dot = jnp.dot(ib, ob, preferred_element_type=jnp.float32)
    dist_ref[...] = jnp.maximum((oo + ii) - 2.0 * dot, 0.0)

    kv_ref[pl.ds(NKEY, NKEY_PAD - NKEY), :] = jnp.full(
        (NKEY_PAD - NKEY, MB), jnp.inf, jnp.float32)
    g8_ref[pl.ds(NG, 1), :] = jnp.full((1, MB), jnp.inf, jnp.float32)
    riota0 = lax.broadcasted_iota(jnp.int32, (MB, MB), 0)

    def grp(g, carry):
        dg = dist_ref[pl.ds(g * MB, MB), :]
        riota = riota0 + g * MB
        vs, cs = [], []
        for t in range(GK):
            vmin = jnp.min(dg, axis=0, keepdims=True)
            sel = jnp.where(dg == vmin, riota, 0x7FFFFFFF)
            vidx = jnp.min(sel, axis=0, keepdims=True)
            vs.append(vmin)
            cs.append(vidx)
            if t < GK - 1:
                dg = jnp.where(riota == vidx, jnp.float32(jnp.inf), dg)
        kv_ref[pl.ds(g * GK, GK), :] = jnp.concatenate(vs, axis=0)
        kc_ref[pl.ds(g * GK, GK), :] = jnp.concatenate(cs, axis=0)
        g8_ref[pl.ds(g, 1), :] = vs[-1]
        return carry

    lax.fori_loop(0, NG, grp, 0)

    v32 = None
    vs, cs = [], []
    for i in range(KN):
        kvv = kv_ref[...]
        vmin = jnp.min(kvv, axis=0, keepdims=True)
        sel = jnp.where(kvv == vmin, kc_ref[...], 0x7FFFFFFF)
        vidx = jnp.min(sel, axis=0, keepdims=True)
        vs.append(vmin)
        cs.append(vidx)
        kv_ref[...] = jnp.where(kc_ref[...] == vidx, jnp.float32(jnp.inf),
                                kvv)
        if (i + 1) % 8 == 0:
            d2_ref[pl.ds(i - 7, 8), :] = jnp.concatenate(vs, axis=0)
            idx_ref[pl.ds(i - 7, 8), :] = jnp.concatenate(cs, axis=0)
            vs, cs = [], []
        v32 = vmin

    ming8 = jnp.min(g8_ref[...], axis=0, keepdims=True)
    need_full = jnp.sum((ming8 <= v32).astype(jnp.float32)) > 0.0

    @pl.when(need_full)
    def _fallback():
        riota_f = lax.broadcasted_iota(jnp.int32, (N_PAD, MB), 0)
        for i in range(KN):
            d = dist_ref[...]
            vmin = jnp.min(d, axis=0, keepdims=True)
            sel = jnp.where(d == vmin, riota_f, 0x7FFFFFFF)
            vidx = jnp.min(sel, axis=0, keepdims=True)
            d2_ref[i:i + 1, :] = vmin
            idx_ref[i:i + 1, :] = vidx
            dist_ref[...] = jnp.where(riota_f == vidx, jnp.float32(jnp.inf),
                                      d)


def _topk_call(op_t, ip_cols):
    return pl.pallas_call(
        _topk_body,
        grid=(M_PAD // MB,),
        in_specs=[
            pl.BlockSpec((8, MB), lambda j: (0, j)),
            pl.BlockSpec((N_PAD, 8), lambda j: (0, 0)),
        ],
        out_specs=[
            pl.BlockSpec((KN, MB), lambda j: (0, j)),
            pl.BlockSpec((KN, MB), lambda j: (0, j)),
        ],
        out_shape=[
            jax.ShapeDtypeStruct((KN, M_PAD), jnp.int32),
            jax.ShapeDtypeStruct((KN, M_PAD), jnp.float32),
        ],
        scratch_shapes=[
            pltpu.VMEM((N_PAD, MB), jnp.float32),
            pltpu.VMEM((NKEY_PAD, MB), jnp.float32),
            pltpu.VMEM((NKEY_PAD, MB), jnp.int32),
            pltpu.VMEM((NG + 1, MB), jnp.float32),
        ],
    )(op_t, ip_cols)


def _sc_gather_body(feat_ref, pos_ref, idx_ref, g_ref, p_ref,
                    idx_v, rows_v, pos_v, sem_f, sem_p):
    wid = lax.axis_index("s") * NC + lax.axis_index("c")

    def body(c, carry):
        r = wid * ROWS_W + c
        pltpu.sync_copy(idx_ref.at[r], idx_v)
        cf = pltpu.async_copy(feat_ref.at[idx_v], rows_v, sem_f)
        cp = pltpu.async_copy(pos_ref.at[idx_v], pos_v, sem_p)
        cf.wait()
        cp.wait()
        off = pl.multiple_of(r * 128, 128)
        pltpu.sync_copy(rows_v, g_ref.at[pl.ds(off, 128)])
        pltpu.sync_copy(pos_v, p_ref.at[pl.ds(off, 128)])
        return carry

    lax.fori_loop(0, ROWS_W, body, 0)


@functools.lru_cache(maxsize=1)
def _sc_gather_kernel():
    return functools.partial(
        pl.kernel,
        mesh=plsc.VectorSubcoreMesh(core_axis_name="c", subcore_axis_name="s"),
        compiler_params=pltpu.CompilerParams(use_tc_tiling_on_sc=False),
        out_type=[
            jax.ShapeDtypeStruct((M_PAD * KN, CIN), jnp.float32),
            jax.ShapeDtypeStruct((M_PAD * KN, 16), jnp.float32),
        ],
        scratch_types=[
            pltpu.VMEM((128,), jnp.int32),
            pltpu.VMEM((128, CIN), jnp.float32),
            pltpu.VMEM((128, 16), jnp.float32),
            pltpu.SemaphoreType.DMA,
            pltpu.SemaphoreType.DMA,
        ],
    )(_sc_gather_body)


def _conv_body(g_ref, px_ref, py_ref, pz_ref, d2_ref, op_ref, kf_ref, b_ref,
               r_ref, out_ref):
    r = r_ref[0, 0]
    relx = px_ref[...] - op_ref[:, 0:1]
    rely = py_ref[...] - op_ref[:, 1:2]
    relz = pz_ref[...] - op_ref[:, 2:3]
    pxn = relx / r
    pyn = rely / r
    pzn = relz / r
    l2 = jnp.sqrt(pxn * pxn + pyn * pyn + pzn * pzn + 1e-12)
    linf = jnp.maximum(jnp.maximum(jnp.abs(pxn), jnp.abs(pyn)), jnp.abs(pzn))
    scale = l2 / jnp.maximum(linf, 1e-8)
    big = linf > 1e-8
    yx = jnp.where(big, pxn * scale, pxn)
    yy = jnp.where(big, pyn * scale, pyn)
    yz = jnp.where(big, pzn * scale, pzn)

    def axis_taps(y):
        co = jnp.clip((y * 0.5 + 0.5) * 2.0, 0.0, 2.0)
        c0 = jnp.floor(co)
        fr = co - c0
        c0i = c0.astype(jnp.int32)
        c1i = jnp.minimum(c0i + 1, 2)
        w0 = 1.0 - fr
        return [
            jnp.where(c0i == t, w0, 0.0) + jnp.where(c1i == t, fr, 0.0)
            for t in (0, 1, 2)
        ]

    ax = axis_taps(yx)
    ay = axis_taps(yy)
    az = axis_taps(yz)
    vm = (d2_ref[...] <= r * r).astype(jnp.float32)
    az = [a * vm for a in az]

    w27 = jnp.stack([ax[f // 9] * ay[(f // 3) % 3] * az[f % 3]
                     for f in range(F3)], axis=1)
    g3 = g_ref[...]
    h3 = jax.lax.dot_general(
        w27, g3, dimension_numbers=(((2,), (1,)), ((0,), (0,))),
        preferred_element_type=jnp.float32)
    acc = jnp.zeros((MB, COUT), jnp.float32)
    for f in range(F3):
        acc = acc + jnp.dot(h3[:, f, :], kf_ref[f * CIN:(f + 1) * CIN, :],
                            preferred_element_type=jnp.float32)

    cnt = jnp.sum(vm, axis=1, keepdims=True)
    inv = jnp.where(cnt > 0, 1.0 / jnp.maximum(cnt, 1.0), 0.0)
    out_ref[...] = acc * inv + b_ref[...]


def _conv_call(g3, px, py, pz, d2, op_pad, kflat, bias2, rad):
    return pl.pallas_call(
        _conv_body,
        grid=(M_PAD // MB,),
        in_specs=[
            pl.BlockSpec((MB, KN, CIN), lambda j: (j, 0, 0)),
            pl.BlockSpec((MB, KN), lambda j: (j, 0)),
            pl.BlockSpec((MB, KN), lambda j: (j, 0)),
            pl.BlockSpec((MB, KN), lambda j: (j, 0)),
            pl.BlockSpec((MB, KN), lambda j: (j, 0)),
            pl.BlockSpec((MB, 16), lambda j: (j, 0)),
            pl.BlockSpec((F3 * CIN, COUT), lambda j: (0, 0)),
            pl.BlockSpec((1, COUT), lambda j: (0, 0)),
            pl.BlockSpec(memory_space=pltpu.SMEM),
        ],
        out_specs=pl.BlockSpec((MB, COUT), lambda j: (j, 0)),
        out_shape=jax.ShapeDtypeStruct((M_PAD, COUT), jnp.float32),
    )(g3, px, py, pz, d2, op_pad, kflat, bias2, rad)


def kernel(inp_features, inp_positions, out_positions, extents, kernel, bias):
    f32 = jnp.float32
    radius = jnp.asarray(extents).astype(f32) * 0.5
    op_t = jnp.zeros((8, M_PAD), f32).at[0:3, :M_OUT].set(
        out_positions.T.astype(f32))
    ip_cols = jnp.zeros((N_PAD, 8), f32)
    ip_cols = ip_cols.at[:N_IN, 0:3].set(inp_positions.astype(f32))
    ip_cols = ip_cols.at[N_IN:, 0:3].set(1e4)
    op_pad = jnp.zeros((M_PAD, 16), f32).at[:M_OUT, 0:3].set(
        out_positions.astype(f32))
    idx_t, d2_t = _topk_call(op_t, ip_cols)
    idx = idx_t.T
    d2 = d2_t.T
    idx2d = idx.reshape(IDX_ROWS, 128)
    pos16 = jnp.zeros((N_IN, 16), f32).at[:, 0:3].set(
        inp_positions.astype(f32))
    g_flat, p_flat = _sc_gather_kernel()(
        inp_features.astype(f32), pos16, idx2d)
    g3 = g_flat.reshape(M_PAD, KN, CIN)
    px = p_flat[:, 0].reshape(M_PAD, KN)
    py = p_flat[:, 1].reshape(M_PAD, KN)
    pz = p_flat[:, 2].reshape(M_PAD, KN)
    kflat = kernel.reshape(F3 * CIN, COUT).astype(f32)
    bias2 = bias.reshape(1, COUT).astype(f32)
    rad = jnp.reshape(radius, (1, 1))
    out_full = _conv_call(g3, px, py, pz, d2, op_pad, kflat, bias2, rad)
    return out_full[:M_OUT]
